# speedup vs baseline: 1.5358x; 1.5358x over previous
import functools

import jax
import jax.numpy as jnp
from jax import lax
from jax.experimental import pallas as pl
from jax.experimental.pallas import tpu as pltpu
from jax.experimental.pallas import tpu_sc as plsc

D_MODEL = 1024
B_TOTAL = 8192
NUM_CORES = 2
NUM_SUBCORES = 16
NW = NUM_CORES * NUM_SUBCORES
B_PER_W = B_TOTAL // NW
CHUNK = 32
NCHUNK = B_PER_W // CHUNK


def _build():
    mesh = plsc.VectorSubcoreMesh(core_axis_name="c", subcore_axis_name="s")

    @functools.partial(
        pl.kernel,
        mesh=mesh,
        out_type=jax.ShapeDtypeStruct((B_TOTAL, D_MODEL), jnp.float32),
        scratch_types=[
            pltpu.VMEM((NCHUNK, CHUNK), jnp.int32),
            pltpu.VMEM((CHUNK, D_MODEL), jnp.float32),
            pltpu.VMEM((CHUNK, D_MODEL), jnp.float32),
            pltpu.SemaphoreType.DMA,
            pltpu.SemaphoreType.DMA,
        ],
    )
    def emb(idx_hbm, table_hbm, out_hbm, idx_v, buf0, buf1, sem0, sem1):
        wid = lax.axis_index("s") * NUM_CORES + lax.axis_index("c")
        base = wid * B_PER_W
        pltpu.sync_copy(idx_hbm.at[wid], idx_v)

        bufs = (buf0, buf1)
        sems = (sem0, sem1)
        copies = [None, None]
        copies[0] = pltpu.async_copy(table_hbm.at[idx_v.at[0]], buf0, sem0)
        for j in range(NCHUNK):
            cur = j % 2
            nxt = 1 - cur
            if j + 1 < NCHUNK:
                copies[nxt] = pltpu.async_copy(
                    table_hbm.at[idx_v.at[j + 1]], bufs[nxt], sems[nxt]
                )
            copies[cur].wait()
            pltpu.sync_copy(
                bufs[cur], out_hbm.at[pl.ds(base + j * CHUNK, CHUNK)]
            )

    return emb


_emb = _build()


@jax.jit
def kernel(input_ids, weight):
    ids = input_ids.reshape(-1).astype(jnp.int32).reshape(NW, NCHUNK, CHUNK)
    out = _emb(ids, weight)
    return out.reshape(input_ids.shape + (D_MODEL,))


# speedup vs baseline: 1.5659x; 1.0196x over previous
import functools

import jax
import jax.numpy as jnp
from jax import lax
from jax.experimental import pallas as pl
from jax.experimental.pallas import tpu as pltpu
from jax.experimental.pallas import tpu_sc as plsc

D_MODEL = 1024
B_TOTAL = 8192
NUM_CORES = 2
NUM_SUBCORES = 16
NW = NUM_CORES * NUM_SUBCORES
B_PER_W = B_TOTAL // NW
CHUNK = 32
NCHUNK = B_PER_W // CHUNK
NBUF = 3


def _build():
    mesh = plsc.VectorSubcoreMesh(core_axis_name="c", subcore_axis_name="s")

    @functools.partial(
        pl.kernel,
        mesh=mesh,
        out_type=jax.ShapeDtypeStruct((B_TOTAL, D_MODEL), jnp.float32),
        scratch_types=[
            pltpu.VMEM((NCHUNK, CHUNK), jnp.int32),
        ]
        + [pltpu.VMEM((CHUNK, D_MODEL), jnp.float32) for _ in range(NBUF)]
        + [pltpu.SemaphoreType.DMA for _ in range(2 * NBUF)],
    )
    def emb(idx_hbm, table_hbm, out_hbm, idx_v, *scratch):
        bufs = scratch[:NBUF]
        gsems = scratch[NBUF:2 * NBUF]
        wsems = scratch[2 * NBUF:]
        wid = lax.axis_index("s") * NUM_CORES + lax.axis_index("c")
        base = wid * B_PER_W
        pltpu.sync_copy(idx_hbm.at[wid], idx_v)

        gathers = [None] * NBUF
        writes = [None] * NBUF
        for j in range(NBUF - 1):
            gathers[j] = pltpu.async_copy(
                table_hbm.at[idx_v.at[j]], bufs[j], gsems[j]
            )
        for j in range(NCHUNK):
            b = j % NBUF
            nj = j + NBUF - 1
            if nj < NCHUNK:
                bn = nj % NBUF
                if writes[bn] is not None:
                    writes[bn].wait()
                gathers[bn] = pltpu.async_copy(
                    table_hbm.at[idx_v.at[nj]], bufs[bn], gsems[bn]
                )
            gathers[b].wait()
            writes[b] = pltpu.async_copy(
                bufs[b], out_hbm.at[pl.ds(base + j * CHUNK, CHUNK)], wsems[b]
            )
        for j in range(NCHUNK - NBUF, NCHUNK):
            writes[j % NBUF].wait()

    return emb


_emb = _build()


@jax.jit
def kernel(input_ids, weight):
    ids = input_ids.reshape(-1).astype(jnp.int32).reshape(NW, NCHUNK, CHUNK)
    out = _emb(ids, weight)
    return out.reshape(input_ids.shape + (D_MODEL,))


# speedup vs baseline: 1.5713x; 1.0035x over previous
import functools

import jax
import jax.numpy as jnp
from jax import lax
from jax.experimental import pallas as pl
from jax.experimental.pallas import tpu as pltpu
from jax.experimental.pallas import tpu_sc as plsc

D_MODEL = 1024
B_TOTAL = 8192
NUM_CORES = 2
NUM_SUBCORES = 16
NW = NUM_CORES * NUM_SUBCORES
B_PER_W = B_TOTAL // NW
CHUNK = 32
NCHUNK = B_PER_W // CHUNK
NBUF = 3


def _build():
    mesh = plsc.VectorSubcoreMesh(core_axis_name="c", subcore_axis_name="s")

    @functools.partial(
        pl.kernel,
        mesh=mesh,
        out_type=jax.ShapeDtypeStruct((B_TOTAL, D_MODEL), jnp.float32),
        scratch_types=[
            pltpu.VMEM((B_PER_W,), jnp.int32),
        ]
        + [pltpu.VMEM((CHUNK, D_MODEL), jnp.float32) for _ in range(NBUF)]
        + [pltpu.SemaphoreType.DMA for _ in range(2 * NBUF)],
    )
    def emb(idx_hbm, table_hbm, out_hbm, idx_v, *scratch):
        bufs = scratch[:NBUF]
        gsems = scratch[NBUF:2 * NBUF]
        wsems = scratch[2 * NBUF:]
        wid = lax.axis_index("s") * NUM_CORES + lax.axis_index("c")
        base = wid * B_PER_W
        pltpu.sync_copy(idx_hbm.at[pl.ds(base, B_PER_W)], idx_v)

        gathers = [None] * NBUF
        writes = [None] * NBUF
        for j in range(NBUF - 1):
            gathers[j] = pltpu.async_copy(
                table_hbm.at[idx_v.at[pl.ds(j * CHUNK, CHUNK)]], bufs[j], gsems[j]
            )
        for j in range(NCHUNK):
            b = j % NBUF
            nj = j + NBUF - 1
            if nj < NCHUNK:
                bn = nj % NBUF
                if writes[bn] is not None:
                    writes[bn].wait()
                gathers[bn] = pltpu.async_copy(
                    table_hbm.at[idx_v.at[pl.ds(nj * CHUNK, CHUNK)]],
                    bufs[bn],
                    gsems[bn],
                )
            gathers[b].wait()
            writes[b] = pltpu.async_copy(
                bufs[b], out_hbm.at[pl.ds(base + j * CHUNK, CHUNK)], wsems[b]
            )
        for j in range(NCHUNK - NBUF, NCHUNK):
            writes[j % NBUF].wait()

    return emb


_emb = _build()


@jax.jit
def kernel(input_ids, weight):
    ids = input_ids.reshape(-1).astype(jnp.int32)
    out = _emb(ids, weight)
    return out.reshape(input_ids.shape + (D_MODEL,))
